# baseline (speedup 1.0000x reference)
import functools

import jax
import jax.numpy as jnp
from jax.experimental import pallas as pl
from jax.experimental.pallas import tpu as pltpu
from jax.experimental.pallas import tpu_sc as plsc

_N = 32768
_D = 768
_E = 64
_TOP_K = 8
_TEMP = 1.0

_BLOCK = 1024

_NC = 2
_NS = 16
_NW = _NC * _NS
_TPW = _N // _NW
_LANES = 16


def _router_body(x_ref, w_ref, scores_ref, weights_ref, idx_ref):
    x = x_ref[...]
    w = w_ref[...]
    s = jax.lax.dot_general(
        x, w, (((1,), (1,)), ((), ())), preferred_element_type=jnp.float32
    )
    scores_ref[...] = s
    st = jax.lax.dot_general(
        w, x, (((1,), (1,)), ((), ())), preferred_element_type=jnp.float32
    )

    expert = jax.lax.broadcasted_iota(jnp.int32, st.shape, 0)
    vals = []
    idxs = []
    for _ in range(_TOP_K):
        top_idx = jnp.argmax(st, axis=0)
        top_val = jnp.max(st, axis=0)
        vals.append(top_val[None, :])
        idxs.append(top_idx[None, :])
        st = jnp.where(expert == top_idx[None, :], -jnp.inf, st)

    top_vals = jnp.concatenate(vals, axis=0)
    top_idxs = jnp.concatenate(idxs, axis=0)
    e = jnp.exp((top_vals - top_vals[0:1, :]) / _TEMP)
    weights_ref[...] = e / jnp.sum(e, axis=0, keepdims=True)
    idx_ref[...] = top_idxs


def _sc_untranspose_body(
    wt_hbm, it_hbm, w_out_hbm, i_out_hbm, w_in, i_in, w_out, i_out
):
    c = jax.lax.axis_index("c")
    s = jax.lax.axis_index("s")
    wid = s * _NC + c
    base = wid * _TPW
    for k in range(_TOP_K):
        pltpu.sync_copy(wt_hbm.at[k, pl.ds(base, _TPW)], w_in.at[pl.ds(k * _TPW, _TPW)])
        pltpu.sync_copy(it_hbm.at[k, pl.ds(base, _TPW)], i_in.at[pl.ds(k * _TPW, _TPW)])

    lane = jax.lax.iota(jnp.int32, _LANES)

    def body(g, carry):
        row0 = g * _LANES
        flat0 = lane * _TOP_K + row0 * _TOP_K
        for k in range(_TOP_K):
            dst = flat0 + k
            wv = w_in[pl.ds(k * _TPW + row0, _LANES)]
            plsc.store_scatter(w_out, [dst], wv)
            iv = i_in[pl.ds(k * _TPW + row0, _LANES)]
            plsc.store_scatter(i_out, [dst], iv)
        return carry

    jax.lax.fori_loop(0, _TPW // _LANES, body, 0)
    pltpu.sync_copy(w_out, w_out_hbm.at[pl.ds(base * _TOP_K, _TPW * _TOP_K)])
    pltpu.sync_copy(i_out, i_out_hbm.at[pl.ds(base * _TOP_K, _TPW * _TOP_K)])


@functools.partial(
    pl.kernel,
    out_type=(
        jax.ShapeDtypeStruct((_N * _TOP_K,), jnp.float32),
        jax.ShapeDtypeStruct((_N * _TOP_K,), jnp.int32),
    ),
    mesh=plsc.VectorSubcoreMesh(core_axis_name="c", subcore_axis_name="s"),
    compiler_params=pltpu.CompilerParams(needs_layout_passes=False),
    scratch_types=[
        pltpu.VMEM((_TOP_K * _TPW,), jnp.float32),
        pltpu.VMEM((_TOP_K * _TPW,), jnp.int32),
        pltpu.VMEM((_TPW * _TOP_K,), jnp.float32),
        pltpu.VMEM((_TPW * _TOP_K,), jnp.int32),
    ],
)
def _sc_untranspose(*args):
    _sc_untranspose_body(*args)


def kernel(x, W):
    grid = (_N // _BLOCK,)
    scores, weights_t, indices_t = pl.pallas_call(
        _router_body,
        grid=grid,
        in_specs=[
            pl.BlockSpec((_BLOCK, _D), lambda i: (i, 0)),
            pl.BlockSpec((_E, _D), lambda i: (0, 0)),
        ],
        out_specs=[
            pl.BlockSpec((_BLOCK, _E), lambda i: (i, 0)),
            pl.BlockSpec((_TOP_K, _BLOCK), lambda i: (0, i)),
            pl.BlockSpec((_TOP_K, _BLOCK), lambda i: (0, i)),
        ],
        out_shape=[
            jax.ShapeDtypeStruct((_N, _E), jnp.float32),
            jax.ShapeDtypeStruct((_TOP_K, _N), jnp.float32),
            jax.ShapeDtypeStruct((_TOP_K, _N), jnp.int32),
        ],
    )(x, W)
    weights_flat, indices_flat = _sc_untranspose(weights_t, indices_t)
    return (
        weights_flat.reshape(_N, _TOP_K),
        indices_flat.reshape(_N, _TOP_K),
        scores,
    )


# speedup vs baseline: 2.2246x; 2.2246x over previous
import functools

import jax
import jax.numpy as jnp
from jax.experimental import pallas as pl
from jax.experimental.pallas import tpu as pltpu
from jax.experimental.pallas import tpu_sc as plsc

_N = 32768
_D = 768
_E = 64
_TOP_K = 8
_TEMP = 1.0

_BLOCK = 2048

_NC = 2
_NS = 16
_NW = _NC * _NS
_TPW = _N // _NW
_LANES = 16


def _router_body(x_ref, w_ref, scores_ref, weights_ref, idx_ref):
    x = x_ref[...]
    w = w_ref[...]
    s = jax.lax.dot_general(
        x, w, (((1,), (1,)), ((), ())), preferred_element_type=jnp.float32
    )
    scores_ref[...] = s
    st = jax.lax.dot_general(
        w, x, (((1,), (1,)), ((), ())), preferred_element_type=jnp.float32
    )

    expert = jax.lax.broadcasted_iota(jnp.int32, st.shape, 0)
    vals = []
    idxs = []
    for _ in range(_TOP_K):
        top_idx = jnp.argmax(st, axis=0)
        top_val = jnp.max(st, axis=0)
        vals.append(top_val[None, :])
        idxs.append(top_idx[None, :])
        st = jnp.where(expert == top_idx[None, :], -jnp.inf, st)

    top_vals = jnp.concatenate(vals, axis=0)
    top_idxs = jnp.concatenate(idxs, axis=0)
    e = jnp.exp((top_vals - top_vals[0:1, :]) / _TEMP)
    weights_ref[...] = e / jnp.sum(e, axis=0, keepdims=True)
    idx_ref[...] = top_idxs


def kernel(x, W):
    grid = (_N // _BLOCK,)
    scores, weights_t, indices_t = pl.pallas_call(
        _router_body,
        grid=grid,
        in_specs=[
            pl.BlockSpec((_BLOCK, _D), lambda i: (i, 0)),
            pl.BlockSpec((_E, _D), lambda i: (0, 0)),
        ],
        out_specs=[
            pl.BlockSpec((_BLOCK, _E), lambda i: (i, 0)),
            pl.BlockSpec((_TOP_K, _BLOCK), lambda i: (0, i)),
            pl.BlockSpec((_TOP_K, _BLOCK), lambda i: (0, i)),
        ],
        out_shape=[
            jax.ShapeDtypeStruct((_N, _E), jnp.float32),
            jax.ShapeDtypeStruct((_TOP_K, _N), jnp.float32),
            jax.ShapeDtypeStruct((_TOP_K, _N), jnp.int32),
        ],
    )(x, W)
    return (weights_t.T, indices_t.T, scores)


# speedup vs baseline: 2.3794x; 1.0696x over previous
import functools

import jax
import jax.numpy as jnp
from jax.experimental import pallas as pl
from jax.experimental.pallas import tpu as pltpu
from jax.experimental.pallas import tpu_sc as plsc

_N = 32768
_D = 768
_E = 64
_TOP_K = 8
_TEMP = 1.0

_BLOCK = 4096

_NC = 2
_NS = 16
_NW = _NC * _NS
_TPW = _N // _NW
_LANES = 16


def _router_body(x_ref, w_ref, scores_ref, weights_ref, idx_ref):
    x = x_ref[...]
    w = w_ref[...]
    s = jax.lax.dot_general(
        x, w, (((1,), (1,)), ((), ())), preferred_element_type=jnp.float32
    )
    scores_ref[...] = s
    st = jax.lax.dot_general(
        w, x, (((1,), (1,)), ((), ())), preferred_element_type=jnp.float32
    )

    expert = jax.lax.broadcasted_iota(jnp.int32, st.shape, 0)
    vals = []
    idxs = []
    for _ in range(_TOP_K):
        top_idx = jnp.argmax(st, axis=0)
        top_val = jnp.max(st, axis=0)
        vals.append(top_val[None, :])
        idxs.append(top_idx[None, :])
        st = jnp.where(expert == top_idx[None, :], -jnp.inf, st)

    top_vals = jnp.concatenate(vals, axis=0)
    top_idxs = jnp.concatenate(idxs, axis=0)
    e = jnp.exp((top_vals - top_vals[0:1, :]) / _TEMP)
    weights_ref[...] = e / jnp.sum(e, axis=0, keepdims=True)
    idx_ref[...] = top_idxs


def kernel(x, W):
    grid = (_N // _BLOCK,)
    scores, weights_t, indices_t = pl.pallas_call(
        _router_body,
        grid=grid,
        in_specs=[
            pl.BlockSpec((_BLOCK, _D), lambda i: (i, 0)),
            pl.BlockSpec((_E, _D), lambda i: (0, 0)),
        ],
        out_specs=[
            pl.BlockSpec((_BLOCK, _E), lambda i: (i, 0)),
            pl.BlockSpec((_TOP_K, _BLOCK), lambda i: (0, i)),
            pl.BlockSpec((_TOP_K, _BLOCK), lambda i: (0, i)),
        ],
        out_shape=[
            jax.ShapeDtypeStruct((_N, _E), jnp.float32),
            jax.ShapeDtypeStruct((_TOP_K, _N), jnp.float32),
            jax.ShapeDtypeStruct((_TOP_K, _N), jnp.int32),
        ],
    )(x, W)
    return (weights_t.T, indices_t.T, scores)
